# Initial kernel scaffold; baseline (speedup 1.0000x reference)
import jax
import jax.numpy as jnp
from jax.experimental import pallas as pl


def kernel(g_dgl, nfeats, efeats, W_msg111, b_msg111, W_msg, b_msg, W_apply, b_apply):
    raise NotImplementedError("write your pallas kernel here")


# speedup vs baseline: 6.2798x; 6.2798x over previous
import functools

import jax
import jax.numpy as jnp
from jax import lax
from jax.experimental import pallas as pl
from jax.experimental.pallas import tpu as pltpu
from jax.experimental.pallas import tpu_sc as plsc

NC = 2
NS = 16
CH = 400
NPAD = 100096
RPT = NPAD // NS
ZCH = RPT // 16
TCB = 4352


def _sc_mesh():
    return plsc.VectorSubcoreMesh(
        core_axis_name="c", subcore_axis_name="s", num_cores=NC,
        num_subcores=NS)


_SC_PARAMS = pltpu.CompilerParams(use_tc_tiling_on_sc=False)


def _edge_scatter(g_dgl, ef2):
    e_total = ef2.shape[0]
    per_tile = e_total // NS
    chunks = per_tile // CH
    zeros = jnp.zeros((ZCH, 16), jnp.float32)

    @functools.partial(
        pl.kernel,
        out_type=jax.ShapeDtypeStruct((NC, NPAD, 16), jnp.float32),
        mesh=_sc_mesh(),
        compiler_params=_SC_PARAMS,
        scratch_types=[
            pltpu.VMEM((CH,), jnp.int32),
            pltpu.VMEM((CH, 16), jnp.float32),
            pltpu.VMEM((ZCH, 16), jnp.float32),
            pltpu.VMEM_SHARED((NPAD, 16), jnp.float32),
        ],
    )
    def k(g_ref, ef_ref, z_ref, se_out, idx_v, feat_v, zv, se_sh):
        c = lax.axis_index("c")
        s = lax.axis_index("s")
        row0 = s * RPT
        pltpu.sync_copy(z_ref, zv)
        for zk in range(16):
            pltpu.sync_copy(zv, se_sh.at[pl.ds(row0 + zk * ZCH, ZCH)])
        plsc.subcore_barrier()
        for cc in range(NC):
            @pl.when(c == cc)
            def _():
                def chunk(ch, carry):
                    base = s * per_tile + ch * CH
                    pltpu.sync_copy(g_ref.at[1].at[pl.ds(base, CH)], idx_v)
                    pltpu.sync_copy(
                        ef_ref.at[pl.ds(base, CH), pl.ds(cc * 16, 16)],
                        feat_v)
                    pltpu.sync_copy(feat_v, se_sh.at[idx_v], add=True)
                    return carry
                lax.fori_loop(0, chunks, chunk, 0)
        plsc.subcore_barrier()
        for cc in range(NC):
            @pl.when(c == cc)
            def _():
                for zk in range(16):
                    r = row0 + zk * ZCH
                    pltpu.sync_copy(se_sh.at[pl.ds(r, ZCH)], zv)
                    pltpu.sync_copy(zv, se_out.at[cc].at[pl.ds(r, ZCH)])

    return k(g_dgl, ef2, zeros)


def _deg_count(g_dgl):
    e_total = g_dgl.shape[1]
    per_core = e_total // NC
    per_tile = per_core // NS
    chunks = per_tile // CH
    zeros = jnp.zeros((ZCH, 16), jnp.float32)
    ones = jnp.ones((CH, 16), jnp.float32)

    @functools.partial(
        pl.kernel,
        out_type=jax.ShapeDtypeStruct((NC, NPAD, 16), jnp.float32),
        mesh=_sc_mesh(),
        compiler_params=_SC_PARAMS,
        scratch_types=[
            pltpu.VMEM((CH,), jnp.int32),
            pltpu.VMEM((CH, 16), jnp.float32),
            pltpu.VMEM((ZCH, 16), jnp.float32),
            pltpu.VMEM_SHARED((NPAD, 16), jnp.float32),
        ],
    )
    def k(g_ref, one_ref, z_ref, deg_out, idx_v, ones_v, zv, deg_sh):
        c = lax.axis_index("c")
        s = lax.axis_index("s")
        row0 = s * RPT
        pltpu.sync_copy(z_ref, zv)
        for zk in range(16):
            pltpu.sync_copy(zv, deg_sh.at[pl.ds(row0 + zk * ZCH, ZCH)])
        pltpu.sync_copy(one_ref, ones_v)
        plsc.subcore_barrier()
        for cc in range(NC):
            @pl.when(c == cc)
            def _():
                def chunk(ch, carry):
                    base = cc * per_core + s * per_tile + ch * CH
                    pltpu.sync_copy(g_ref.at[1].at[pl.ds(base, CH)], idx_v)
                    pltpu.sync_copy(ones_v, deg_sh.at[idx_v], add=True)
                    return carry
                lax.fori_loop(0, chunks, chunk, 0)
        plsc.subcore_barrier()
        for cc in range(NC):
            @pl.when(c == cc)
            def _():
                for zk in range(16):
                    r = row0 + zk * ZCH
                    pltpu.sync_copy(deg_sh.at[pl.ds(r, ZCH)], zv)
                    pltpu.sync_copy(zv, deg_out.at[cc].at[pl.ds(r, ZCH)])

    return k(g_dgl, ones, zeros)


def _gather_scatter(g_dgl, h_parts):
    e_total = g_dgl.shape[1]
    per_tile = e_total // NS
    chunks = per_tile // CH
    zeros = jnp.zeros((ZCH, 16), jnp.float32)

    @functools.partial(
        pl.kernel,
        out_type=jax.ShapeDtypeStruct((NC, NPAD, 16), jnp.float32),
        mesh=_sc_mesh(),
        compiler_params=_SC_PARAMS,
        scratch_types=[
            pltpu.VMEM((CH,), jnp.int32),
            pltpu.VMEM((CH,), jnp.int32),
            pltpu.VMEM((CH, 16), jnp.float32),
            pltpu.VMEM((ZCH, 16), jnp.float32),
            pltpu.VMEM_SHARED((NPAD, 16), jnp.float32),
        ],
    )
    def k(g_ref, h_ref, z_ref, sh_out, sidx_v, didx_v, rows_v, zv, sh_sh):
        c = lax.axis_index("c")
        s = lax.axis_index("s")
        row0 = s * RPT
        pltpu.sync_copy(z_ref, zv)
        for zk in range(16):
            pltpu.sync_copy(zv, sh_sh.at[pl.ds(row0 + zk * ZCH, ZCH)])
        plsc.subcore_barrier()
        for cc in range(NC):
            @pl.when(c == cc)
            def _():
                def chunk(ch, carry):
                    base = s * per_tile + ch * CH
                    pltpu.sync_copy(g_ref.at[0].at[pl.ds(base, CH)], sidx_v)
                    pltpu.sync_copy(g_ref.at[1].at[pl.ds(base, CH)], didx_v)
                    pltpu.sync_copy(h_ref.at[cc].at[sidx_v], rows_v)
                    pltpu.sync_copy(rows_v, sh_sh.at[didx_v], add=True)
                    return carry
                lax.fori_loop(0, chunks, chunk, 0)
        plsc.subcore_barrier()
        for cc in range(NC):
            @pl.when(c == cc)
            def _():
                for zk in range(16):
                    r = row0 + zk * ZCH
                    pltpu.sync_copy(sh_sh.at[pl.ds(r, ZCH)], zv)
                    pltpu.sync_copy(zv, sh_out.at[cc].at[pl.ds(r, ZCH)])

    return k(g_dgl, h_parts, zeros)


def _tc_stage1(se_parts, deg_parts, w1_parts, b1_parts):
    nb = NPAD // TCB

    def body(se_ref, deg_ref, w_ref, b_ref, out_ref):
        se = se_ref[...]
        d = deg_ref[0, :, 0] + deg_ref[1, :, 0]
        inv2 = 2.0 / jnp.maximum(d, 1.0)
        mask = jnp.minimum(d, 1.0)
        w = w_ref[0]
        x_lo = se[0] * inv2[:, None]
        x_hi = se[1] * inv2[:, None]
        acc = (lax.dot_general(x_lo, w[:16], (((1,), (0,)), ((), ())),
                               preferred_element_type=jnp.float32)
               + lax.dot_general(x_hi, w[16:], (((1,), (0,)), ((), ())),
                                 preferred_element_type=jnp.float32))
        h = jnp.maximum(acc + b_ref[0, 0] * mask[:, None], 0.0)
        out_ref[0] = h

    return pl.pallas_call(
        body,
        grid=(NC, nb),
        in_specs=[
            pl.BlockSpec((NC, TCB, 16), lambda c, i: (0, i, 0)),
            pl.BlockSpec((NC, TCB, 16), lambda c, i: (0, i, 0)),
            pl.BlockSpec((1, 32, 16), lambda c, i: (c, 0, 0)),
            pl.BlockSpec((1, 1, 16), lambda c, i: (c, 0, 0)),
        ],
        out_specs=pl.BlockSpec((1, TCB, 16), lambda c, i: (c, i, 0)),
        out_shape=jax.ShapeDtypeStruct((NC, NPAD, 16), jnp.float32),
    )(se_parts, deg_parts, w1_parts, b1_parts)


def _tc_stage2(sh_parts, se_parts, deg_parts, M_h, M_e, consts):
    nb = NPAD // TCB

    def body(sh_ref, se_ref, deg_ref, mh_ref, me_ref, c_ref, out_ref):
        sh = sh_ref[...]
        se = se_ref[...]
        d = deg_ref[0, :, 0] + deg_ref[1, :, 0]
        inv = 1.0 / jnp.maximum(d, 1.0)
        mask = jnp.minimum(d, 1.0)
        mh = mh_ref[...]
        me = me_ref[...]
        dn = (((1,), (0,)), ((), ()))
        acc = (lax.dot_general(sh[0], mh[:16], dn,
                               preferred_element_type=jnp.float32)
               + lax.dot_general(sh[1], mh[16:], dn,
                                 preferred_element_type=jnp.float32)
               + lax.dot_general(se[0], me[:16], dn,
                                 preferred_element_type=jnp.float32)
               + lax.dot_general(se[1], me[16:], dn,
                                 preferred_element_type=jnp.float32))
        cvec = c_ref[...]
        out = acc * inv[:, None] + mask[:, None] * cvec[0] + cvec[1]
        out_ref[...] = jnp.maximum(out, 0.0)

    return pl.pallas_call(
        body,
        grid=(nb,),
        in_specs=[
            pl.BlockSpec((NC, TCB, 16), lambda i: (0, i, 0)),
            pl.BlockSpec((NC, TCB, 16), lambda i: (0, i, 0)),
            pl.BlockSpec((NC, TCB, 16), lambda i: (0, i, 0)),
            pl.BlockSpec((32, 128), lambda i: (0, 0)),
            pl.BlockSpec((32, 128), lambda i: (0, 0)),
            pl.BlockSpec((2, 128), lambda i: (0, 0)),
        ],
        out_specs=pl.BlockSpec((TCB, 128), lambda i: (i, 0)),
        out_shape=jax.ShapeDtypeStruct((NPAD, 128), jnp.float32),
    )(sh_parts, se_parts, deg_parts, M_h, M_e, consts)


def kernel(g_dgl, nfeats, efeats, W_msg111, b_msg111, W_msg, b_msg,
           W_apply, b_apply):
    n_nodes = nfeats.shape[0]
    e_total = g_dgl.shape[1]
    ef2 = efeats.reshape(e_total, 32)

    se_parts = _edge_scatter(g_dgl, ef2)
    deg_parts = _deg_count(g_dgl)

    w1_parts = W_msg111.T.reshape(32, NC, 16).swapaxes(0, 1)
    h_parts = _tc_stage1(se_parts, deg_parts, w1_parts,
                         b_msg111.reshape(NC, 1, 16))

    sh_parts = _gather_scatter(g_dgl, h_parts)

    M_h = 2.0 * (W_msg[:, :32].T @ W_apply.T)
    M_e = 2.0 * (W_msg[:, 32:].T @ W_apply.T)
    consts = jnp.stack([2.0 * (W_apply @ b_msg), b_apply])
    out = _tc_stage2(sh_parts, se_parts, deg_parts, M_h, M_e, consts)
    return out[:n_nodes].reshape(n_nodes, 1, 128)


# speedup vs baseline: 7.7826x; 1.2393x over previous
import functools

import jax
import jax.numpy as jnp
from jax import lax
from jax.experimental import pallas as pl
from jax.experimental.pallas import tpu as pltpu
from jax.experimental.pallas import tpu_sc as plsc

NC = 2
NS = 16
CH = 400
NPAD = 100096
RPT = NPAD // NS
ZCH = RPT // 16
TCB = 4352


def _sc_mesh():
    return plsc.VectorSubcoreMesh(
        core_axis_name="c", subcore_axis_name="s", num_cores=NC,
        num_subcores=NS)


_SC_PARAMS = pltpu.CompilerParams(use_tc_tiling_on_sc=False)


def _edge_scatter(g_dgl, ef2):
    e_total = ef2.shape[0]
    per_tile = e_total // NS
    chunks = per_tile // CH
    zeros = jnp.zeros((ZCH, 16), jnp.float32)

    @functools.partial(
        pl.kernel,
        out_type=jax.ShapeDtypeStruct((NC, NPAD, 16), jnp.float32),
        mesh=_sc_mesh(),
        compiler_params=_SC_PARAMS,
        scratch_types=[
            pltpu.VMEM((CH,), jnp.int32),
            pltpu.VMEM((CH,), jnp.int32),
            pltpu.VMEM((CH, 16), jnp.float32),
            pltpu.VMEM((CH, 16), jnp.float32),
            pltpu.VMEM((ZCH, 16), jnp.float32),
            pltpu.VMEM_SHARED((NPAD, 16), jnp.float32),
            pltpu.SemaphoreType.DMA,
            pltpu.SemaphoreType.DMA,
        ],
    )
    def k(g_ref, ef_ref, z_ref, se_out, idx0, idx1, feat0, feat1, zv,
          se_sh, lsem0, lsem1):
        c = lax.axis_index("c")
        s = lax.axis_index("s")
        row0 = s * RPT
        pltpu.sync_copy(z_ref, zv)
        for zk in range(16):
            pltpu.sync_copy(zv, se_sh.at[pl.ds(row0 + zk * ZCH, ZCH)])
        plsc.subcore_barrier()
        for cc in range(NC):
            @pl.when(c == cc)
            def _():
                def sl(ch, ib, fb, sem):
                    base = s * per_tile + ch * CH
                    pltpu.async_copy(
                        g_ref.at[1].at[pl.ds(base, CH)], ib, sem)
                    pltpu.async_copy(
                        ef_ref.at[pl.ds(base, CH), pl.ds(cc * 16, 16)],
                        fb, sem)

                def wl(ib, fb, sem):
                    pltpu.make_async_copy(
                        g_ref.at[1].at[pl.ds(0, CH)], ib, sem).wait()
                    pltpu.make_async_copy(
                        ef_ref.at[pl.ds(0, CH), pl.ds(cc * 16, 16)],
                        fb, sem).wait()

                sl(0, idx0, feat0, lsem0)

                def body(gg, carry):
                    ch0 = 2 * gg
                    wl(idx0, feat0, lsem0)
                    sl(ch0 + 1, idx1, feat1, lsem1)
                    pltpu.sync_copy(feat0, se_sh.at[idx0], add=True)
                    wl(idx1, feat1, lsem1)
                    @pl.when(ch0 + 2 < chunks)
                    def _():
                        sl(ch0 + 2, idx0, feat0, lsem0)
                    pltpu.sync_copy(feat1, se_sh.at[idx1], add=True)
                    return carry
                lax.fori_loop(0, chunks // 2, body, 0)
        plsc.subcore_barrier()
        for cc in range(NC):
            @pl.when(c == cc)
            def _():
                for zk in range(16):
                    r = row0 + zk * ZCH
                    pltpu.sync_copy(se_sh.at[pl.ds(r, ZCH)], zv)
                    pltpu.sync_copy(zv, se_out.at[cc].at[pl.ds(r, ZCH)])

    return k(g_dgl, ef2, zeros)


def _deg_count(g_dgl):
    e_total = g_dgl.shape[1]
    per_core = e_total // NC
    per_tile = per_core // NS
    chunks = per_tile // CH
    zeros = jnp.zeros((ZCH, 16), jnp.float32)
    ones = jnp.ones((CH, 16), jnp.float32)

    @functools.partial(
        pl.kernel,
        out_type=jax.ShapeDtypeStruct((NC, NPAD, 16), jnp.float32),
        mesh=_sc_mesh(),
        compiler_params=_SC_PARAMS,
        scratch_types=[
            pltpu.VMEM((CH,), jnp.int32),
            pltpu.VMEM((CH,), jnp.int32),
            pltpu.VMEM((CH, 16), jnp.float32),
            pltpu.VMEM((ZCH, 16), jnp.float32),
            pltpu.VMEM_SHARED((NPAD, 16), jnp.float32),
            pltpu.SemaphoreType.DMA,
            pltpu.SemaphoreType.DMA,
        ],
    )
    def k(g_ref, one_ref, z_ref, deg_out, idx0, idx1, ones_v, zv, deg_sh,
          lsem0, lsem1):
        c = lax.axis_index("c")
        s = lax.axis_index("s")
        row0 = s * RPT
        pltpu.sync_copy(z_ref, zv)
        for zk in range(16):
            pltpu.sync_copy(zv, deg_sh.at[pl.ds(row0 + zk * ZCH, ZCH)])
        pltpu.sync_copy(one_ref, ones_v)
        plsc.subcore_barrier()
        for cc in range(NC):
            @pl.when(c == cc)
            def _():
                def sl(ch, ib, sem):
                    base = cc * per_core + s * per_tile + ch * CH
                    pltpu.async_copy(
                        g_ref.at[1].at[pl.ds(base, CH)], ib, sem)

                def wl(ib, sem):
                    pltpu.make_async_copy(
                        g_ref.at[1].at[pl.ds(0, CH)], ib, sem).wait()

                sl(0, idx0, lsem0)

                def body(gg, carry):
                    ch0 = 2 * gg
                    wl(idx0, lsem0)
                    sl(ch0 + 1, idx1, lsem1)
                    pltpu.sync_copy(ones_v, deg_sh.at[idx0], add=True)
                    wl(idx1, lsem1)
                    @pl.when(ch0 + 2 < chunks)
                    def _():
                        sl(ch0 + 2, idx0, lsem0)
                    pltpu.sync_copy(ones_v, deg_sh.at[idx1], add=True)
                    return carry
                lax.fori_loop(0, chunks // 2, body, 0)
                if chunks % 2:
                    wl(idx0, lsem0)
                    pltpu.sync_copy(ones_v, deg_sh.at[idx0], add=True)
        plsc.subcore_barrier()
        for cc in range(NC):
            @pl.when(c == cc)
            def _():
                for zk in range(16):
                    r = row0 + zk * ZCH
                    pltpu.sync_copy(deg_sh.at[pl.ds(r, ZCH)], zv)
                    pltpu.sync_copy(zv, deg_out.at[cc].at[pl.ds(r, ZCH)])

    return k(g_dgl, ones, zeros)


def _gather_scatter(g_dgl, h_parts):
    e_total = g_dgl.shape[1]
    per_tile = e_total // NS
    chunks = per_tile // CH
    zeros = jnp.zeros((ZCH, 16), jnp.float32)

    @functools.partial(
        pl.kernel,
        out_type=jax.ShapeDtypeStruct((NC, NPAD, 16), jnp.float32),
        mesh=_sc_mesh(),
        compiler_params=_SC_PARAMS,
        scratch_types=[
            pltpu.VMEM((CH,), jnp.int32),
            pltpu.VMEM((CH,), jnp.int32),
            pltpu.VMEM((CH,), jnp.int32),
            pltpu.VMEM((CH,), jnp.int32),
            pltpu.VMEM((CH, 16), jnp.float32),
            pltpu.VMEM((ZCH, 16), jnp.float32),
            pltpu.VMEM_SHARED((NPAD, 16), jnp.float32),
            pltpu.SemaphoreType.DMA,
            pltpu.SemaphoreType.DMA,
        ],
    )
    def k(g_ref, h_ref, z_ref, sh_out, sidx0, sidx1, didx0, didx1,
          rows_v, zv, sh_sh, lsem0, lsem1):
        c = lax.axis_index("c")
        s = lax.axis_index("s")
        row0 = s * RPT
        pltpu.sync_copy(z_ref, zv)
        for zk in range(16):
            pltpu.sync_copy(zv, sh_sh.at[pl.ds(row0 + zk * ZCH, ZCH)])
        plsc.subcore_barrier()
        for cc in range(NC):
            @pl.when(c == cc)
            def _():
                def sl(ch, sb, db, sem):
                    base = s * per_tile + ch * CH
                    pltpu.async_copy(
                        g_ref.at[0].at[pl.ds(base, CH)], sb, sem)
                    pltpu.async_copy(
                        g_ref.at[1].at[pl.ds(base, CH)], db, sem)

                def wl(sb, db, sem):
                    pltpu.make_async_copy(
                        g_ref.at[0].at[pl.ds(0, CH)], sb, sem).wait()
                    pltpu.make_async_copy(
                        g_ref.at[1].at[pl.ds(0, CH)], db, sem).wait()

                sl(0, sidx0, didx0, lsem0)

                def body(gg, carry):
                    ch0 = 2 * gg
                    wl(sidx0, didx0, lsem0)
                    sl(ch0 + 1, sidx1, didx1, lsem1)
                    pltpu.sync_copy(h_ref.at[cc].at[sidx0], rows_v)
                    pltpu.sync_copy(rows_v, sh_sh.at[didx0], add=True)
                    wl(sidx1, didx1, lsem1)
                    @pl.when(ch0 + 2 < chunks)
                    def _():
                        sl(ch0 + 2, sidx0, didx0, lsem0)
                    pltpu.sync_copy(h_ref.at[cc].at[sidx1], rows_v)
                    pltpu.sync_copy(rows_v, sh_sh.at[didx1], add=True)
                    return carry
                lax.fori_loop(0, chunks // 2, body, 0)
        plsc.subcore_barrier()
        for cc in range(NC):
            @pl.when(c == cc)
            def _():
                for zk in range(16):
                    r = row0 + zk * ZCH
                    pltpu.sync_copy(sh_sh.at[pl.ds(r, ZCH)], zv)
                    pltpu.sync_copy(zv, sh_out.at[cc].at[pl.ds(r, ZCH)])

    return k(g_dgl, h_parts, zeros)


def _tc_stage1(se_parts, deg_parts, w1_parts, b1_parts):
    nb = NPAD // TCB

    def body(se_ref, deg_ref, w_ref, b_ref, out_ref):
        se = se_ref[...]
        d = deg_ref[0, :, 0] + deg_ref[1, :, 0]
        inv2 = 2.0 / jnp.maximum(d, 1.0)
        mask = jnp.minimum(d, 1.0)
        w = w_ref[0]
        x_lo = se[0] * inv2[:, None]
        x_hi = se[1] * inv2[:, None]
        acc = (lax.dot_general(x_lo, w[:16], (((1,), (0,)), ((), ())),
                               preferred_element_type=jnp.float32)
               + lax.dot_general(x_hi, w[16:], (((1,), (0,)), ((), ())),
                                 preferred_element_type=jnp.float32))
        h = jnp.maximum(acc + b_ref[0, 0] * mask[:, None], 0.0)
        out_ref[0] = h

    return pl.pallas_call(
        body,
        grid=(NC, nb),
        in_specs=[
            pl.BlockSpec((NC, TCB, 16), lambda c, i: (0, i, 0)),
            pl.BlockSpec((NC, TCB, 16), lambda c, i: (0, i, 0)),
            pl.BlockSpec((1, 32, 16), lambda c, i: (c, 0, 0)),
            pl.BlockSpec((1, 1, 16), lambda c, i: (c, 0, 0)),
        ],
        out_specs=pl.BlockSpec((1, TCB, 16), lambda c, i: (c, i, 0)),
        out_shape=jax.ShapeDtypeStruct((NC, NPAD, 16), jnp.float32),
    )(se_parts, deg_parts, w1_parts, b1_parts)


def _tc_stage2(sh_parts, se_parts, deg_parts, M_h, M_e, consts):
    nb = NPAD // TCB

    def body(sh_ref, se_ref, deg_ref, mh_ref, me_ref, c_ref, out_ref):
        sh = sh_ref[...]
        se = se_ref[...]
        d = deg_ref[0, :, 0] + deg_ref[1, :, 0]
        inv = 1.0 / jnp.maximum(d, 1.0)
        mask = jnp.minimum(d, 1.0)
        mh = mh_ref[...]
        me = me_ref[...]
        dn = (((1,), (0,)), ((), ()))
        acc = (lax.dot_general(sh[0], mh[:16], dn,
                               preferred_element_type=jnp.float32)
               + lax.dot_general(sh[1], mh[16:], dn,
                                 preferred_element_type=jnp.float32)
               + lax.dot_general(se[0], me[:16], dn,
                                 preferred_element_type=jnp.float32)
               + lax.dot_general(se[1], me[16:], dn,
                                 preferred_element_type=jnp.float32))
        cvec = c_ref[...]
        out = acc * inv[:, None] + mask[:, None] * cvec[0] + cvec[1]
        out_ref[...] = jnp.maximum(out, 0.0)

    return pl.pallas_call(
        body,
        grid=(nb,),
        in_specs=[
            pl.BlockSpec((NC, TCB, 16), lambda i: (0, i, 0)),
            pl.BlockSpec((NC, TCB, 16), lambda i: (0, i, 0)),
            pl.BlockSpec((NC, TCB, 16), lambda i: (0, i, 0)),
            pl.BlockSpec((32, 128), lambda i: (0, 0)),
            pl.BlockSpec((32, 128), lambda i: (0, 0)),
            pl.BlockSpec((2, 128), lambda i: (0, 0)),
        ],
        out_specs=pl.BlockSpec((TCB, 128), lambda i: (i, 0)),
        out_shape=jax.ShapeDtypeStruct((NPAD, 128), jnp.float32),
    )(sh_parts, se_parts, deg_parts, M_h, M_e, consts)


def kernel(g_dgl, nfeats, efeats, W_msg111, b_msg111, W_msg, b_msg,
           W_apply, b_apply):
    n_nodes = nfeats.shape[0]
    e_total = g_dgl.shape[1]
    ef2 = efeats.reshape(e_total, 32)

    se_parts = _edge_scatter(g_dgl, ef2)
    deg_parts = _deg_count(g_dgl)

    w1_parts = W_msg111.T.reshape(32, NC, 16).swapaxes(0, 1)
    h_parts = _tc_stage1(se_parts, deg_parts, w1_parts,
                         b_msg111.reshape(NC, 1, 16))

    sh_parts = _gather_scatter(g_dgl, h_parts)

    M_h = 2.0 * (W_msg[:, :32].T @ W_apply.T)
    M_e = 2.0 * (W_msg[:, 32:].T @ W_apply.T)
    consts = jnp.stack([2.0 * (W_apply @ b_msg), b_apply])
    out = _tc_stage2(sh_parts, se_parts, deg_parts, M_h, M_e, consts)
    return out[:n_nodes].reshape(n_nodes, 1, 128)
